# baseline (speedup 1.0000x reference)
import jax
import jax.numpy as jnp
from jax.experimental import pallas as pl
from jax.experimental.pallas import tpu as pltpu

NUM_ANCHORS = 4
NUM_LABELS = 5
BLK = 2048


def _fused_heads_kernel(x_ref, wc_ref, bc_ref, wl_ref, bl_ref, wr_ref, br_ref,
                        conf_ref, cls_ref, reg_ref):
    x = x_ref[...]
    conf_ref[...] = (
        jnp.dot(x, wc_ref[...], preferred_element_type=jnp.float32) + bc_ref[...]
    )
    cls_ref[...] = (
        jnp.dot(x, wl_ref[...], preferred_element_type=jnp.float32) + bl_ref[...]
    )
    reg_ref[...] = (
        jnp.dot(x, wr_ref[...], preferred_element_type=jnp.float32) + br_ref[...]
    )


def kernel(hidden_states, W_conf, b_conf, W_cls, b_cls, W_reg, b_reg):
    bsz, seq_len, hidden = hidden_states.shape
    x = hidden_states.reshape(bsz * seq_len, hidden)
    n = bsz * seq_len
    nc, nl, nr = NUM_ANCHORS, NUM_ANCHORS * NUM_LABELS, NUM_ANCHORS * 2

    def const_spec(r, c):
        return pl.BlockSpec((r, c), lambda i: (0, 0))

    conf, cls_, reg = pl.pallas_call(
        _fused_heads_kernel,
        grid=(n // BLK,),
        in_specs=[
            pl.BlockSpec((BLK, hidden), lambda i: (i, 0)),
            const_spec(hidden, nc), const_spec(1, nc),
            const_spec(hidden, nl), const_spec(1, nl),
            const_spec(hidden, nr), const_spec(1, nr),
        ],
        out_specs=[
            pl.BlockSpec((BLK, nc), lambda i: (i, 0)),
            pl.BlockSpec((BLK, nl), lambda i: (i, 0)),
            pl.BlockSpec((BLK, nr), lambda i: (i, 0)),
        ],
        out_shape=[
            jax.ShapeDtypeStruct((n, nc), jnp.float32),
            jax.ShapeDtypeStruct((n, nl), jnp.float32),
            jax.ShapeDtypeStruct((n, nr), jnp.float32),
        ],
        compiler_params=pltpu.CompilerParams(
            dimension_semantics=("parallel",),
        ),
    )(x, W_conf, b_conf.reshape(1, nc), W_cls, b_cls.reshape(1, nl),
      W_reg, b_reg.reshape(1, nr))

    return (
        conf.reshape(bsz, seq_len, NUM_ANCHORS),
        cls_.reshape(bsz, seq_len, NUM_ANCHORS, NUM_LABELS),
        reg.reshape(bsz, seq_len, NUM_ANCHORS, 2),
    )


# speedup vs baseline: 1.0944x; 1.0944x over previous
import jax
import jax.numpy as jnp
from jax.experimental import pallas as pl
from jax.experimental.pallas import tpu as pltpu

NUM_ANCHORS = 4
NUM_LABELS = 5
NC = NUM_ANCHORS
NL = NUM_ANCHORS * NUM_LABELS
NR = NUM_ANCHORS * 2
BLK = 4096


def _fused_heads_kernel(x_ref, wc_ref, bc_ref, wl_ref, bl_ref, wr_ref, br_ref,
                        conf_ref, cls_ref, reg_ref, w_scr):
    @pl.when(pl.program_id(0) == 0)
    def _():
        w_scr[:, :NC] = wc_ref[...]
        w_scr[:, NC:NC + NL] = wl_ref[...]
        w_scr[:, NC + NL:] = wr_ref[...]

    acc = jnp.dot(x_ref[...], w_scr[...], preferred_element_type=jnp.float32)
    conf_ref[...] = acc[:, :NC] + bc_ref[...]
    cls_ref[...] = acc[:, NC:NC + NL] + bl_ref[...]
    reg_ref[...] = acc[:, NC + NL:] + br_ref[...]


def kernel(hidden_states, W_conf, b_conf, W_cls, b_cls, W_reg, b_reg):
    bsz, seq_len, hidden = hidden_states.shape
    x = hidden_states.reshape(bsz * seq_len, hidden)
    n = bsz * seq_len

    def const_spec(r, c):
        return pl.BlockSpec((r, c), lambda i: (0, 0))

    conf, cls_, reg = pl.pallas_call(
        _fused_heads_kernel,
        grid=(n // BLK,),
        in_specs=[
            pl.BlockSpec((BLK, hidden), lambda i: (i, 0)),
            const_spec(hidden, NC), const_spec(1, NC),
            const_spec(hidden, NL), const_spec(1, NL),
            const_spec(hidden, NR), const_spec(1, NR),
        ],
        out_specs=[
            pl.BlockSpec((BLK, NC), lambda i: (i, 0)),
            pl.BlockSpec((BLK, NL), lambda i: (i, 0)),
            pl.BlockSpec((BLK, NR), lambda i: (i, 0)),
        ],
        out_shape=[
            jax.ShapeDtypeStruct((n, NC), jnp.float32),
            jax.ShapeDtypeStruct((n, NL), jnp.float32),
            jax.ShapeDtypeStruct((n, NR), jnp.float32),
        ],
        scratch_shapes=[pltpu.VMEM((hidden, NC + NL + NR), jnp.float32)],
        compiler_params=pltpu.CompilerParams(
            dimension_semantics=("arbitrary",),
        ),
    )(x, W_conf, b_conf.reshape(1, NC), W_cls, b_cls.reshape(1, NL),
      W_reg, b_reg.reshape(1, NR))

    return (
        conf.reshape(bsz, seq_len, NUM_ANCHORS),
        cls_.reshape(bsz, seq_len, NUM_ANCHORS, NUM_LABELS),
        reg.reshape(bsz, seq_len, NUM_ANCHORS, 2),
    )
